# baseline (speedup 1.0000x reference)
import functools

import jax
import jax.numpy as jnp
from jax import lax
from jax.experimental import pallas as pl
from jax.experimental.pallas import tpu as pltpu
from jax.experimental.pallas import tpu_sc as plsc

TRAJ = 16384
EMB = 64
HEMB = EMB // 2
VOCAB = 100000
NC = 2
NS = 16
NW = NC * NS
ROWS_PER_W = TRAJ // NW
CHUNK = 128
NCHUNK = ROWS_PER_W // CHUNK

SUB = 512
GROUP = 4 * SUB
TGRID = 49
PACKED_ROWS = TGRID * SUB
PACKED_FLAT = 4 * PACKED_ROWS


def _lrelu(y):
    return jnp.where(y >= 0, y, 0.01 * y)


def _rtne_bf16_bits(x):
    b = lax.bitcast_convert_type(x, jnp.uint32)
    return b + jnp.uint32(0x7FFF) + ((b >> 16) & jnp.uint32(1))


def _pack_bf16(ylo, yhi):
    lo = _rtne_bf16_bits(ylo) >> 16
    hi = _rtne_bf16_bits(yhi) & jnp.uint32(0xFFFF0000)
    return lax.bitcast_convert_type(lo | hi, jnp.float32)


def _tc_pack(tableT):
    def body(t0, t1, t2, t3, o_ref):
        row = lax.broadcasted_iota(jnp.int32, (4 * EMB, 2 * EMB), 0)
        col = lax.broadcasted_iota(jnp.int32, (4 * EMB, 2 * EMB), 1)
        same_sub = (row >> 6) == (col >> 5)
        d = row & 63
        j = col & 31
        sel_lo = (same_sub & (d == j)).astype(jnp.float32)
        sel_hi = (same_sub & (d == j + HEMB)).astype(jnp.float32)
        t = jnp.concatenate([t0[...], t1[...], t2[...], t3[...]], axis=0)
        ylo = lax.dot_general(t, sel_lo, (((0,), (0,)), ((), ())),
                              preferred_element_type=jnp.float32)
        yhi = lax.dot_general(t, sel_hi, (((0,), (0,)), ((), ())),
                              preferred_element_type=jnp.float32)
        o_ref[...] = _pack_bf16(ylo, yhi)

    return pl.pallas_call(
        body,
        grid=(TGRID,),
        in_specs=[
            pl.BlockSpec((EMB, SUB), lambda i, s=s: (0, 4 * i + s))
            for s in range(4)
        ],
        out_specs=pl.BlockSpec((SUB, 4 * HEMB), lambda i: (i, 0)),
        out_shape=jax.ShapeDtypeStruct((2 * PACKED_ROWS, 4 * HEMB), jnp.float32),
    )(tableT, tableT, tableT, tableT)


def _sc_gather(idx2d, packed_flat):
    mesh = plsc.VectorSubcoreMesh(core_axis_name="c", subcore_axis_name="s")

    @functools.partial(
        pl.kernel,
        mesh=mesh,
        out_type=jax.ShapeDtypeStruct((8 * TRAJ, 2 * EMB), jnp.float32),
        scratch_types=[
            pltpu.VMEM((NCHUNK, CHUNK), jnp.int32),
            pltpu.VMEM((NCHUNK, CHUNK), jnp.int32),
            pltpu.VMEM((ROWS_PER_W, HEMB), jnp.float32),
            pltpu.SemaphoreType.DMA,
        ],
        compiler_params=pltpu.CompilerParams(use_tc_tiling_on_sc=False),
    )
    def k(idx_hbm, table_hbm, out_hbm, idx_raw, idx_t, rows_v, sem):
        wid = lax.axis_index("s") * NC + lax.axis_index("c")
        base = wid * ROWS_PER_W
        pltpu.sync_copy(idx_hbm.at[pl.ds(wid * NCHUNK, NCHUNK)], idx_raw)
        for c in range(NCHUNK):
            for kk in range(CHUNK // 16):
                v = idx_raw[c, pl.ds(kk * 16, 16)]
                grp = jnp.left_shift(jnp.right_shift(v, 11), 11)
                loc = jnp.left_shift(jnp.bitwise_and(v, jnp.int32(SUB - 1)), 2)
                sub = jnp.bitwise_and(jnp.right_shift(v, 9), jnp.int32(3))
                idx_t[c, pl.ds(kk * 16, 16)] = grp + loc + sub
        copies = [
            pltpu.make_async_copy(
                table_hbm.at[idx_t.at[c]],
                rows_v.at[pl.ds(c * CHUNK, CHUNK)],
                sem,
            )
            for c in range(NCHUNK)
        ]
        for cp in copies:
            cp.start()
        for cp in copies:
            cp.wait()
        pltpu.sync_copy(
            rows_v, out_hbm.at[pl.ds(base, ROWS_PER_W), pl.ds(0, HEMB)]
        )

    return k(idx2d, packed_flat)


def _tc_linear(g, W, b2d):
    def body(g_ref, w_ref, b_ref, o_ref):
        xi = lax.bitcast_convert_type(g_ref[:, 0:HEMB], jnp.uint32)
        xlo = lax.bitcast_convert_type(xi << 16, jnp.float32)
        xhi = lax.bitcast_convert_type(xi & jnp.uint32(0xFFFF0000), jnp.float32)
        ylo = lax.dot_general(w_ref[:, 0:HEMB], xlo, (((1,), (1,)), ((), ())),
                              preferred_element_type=jnp.float32)
        yhi = lax.dot_general(w_ref[:, HEMB:EMB], xhi, (((1,), (1,)), ((), ())),
                              preferred_element_type=jnp.float32)
        o_ref[0] = _lrelu(ylo + yhi + b_ref[...].T)

    blk = 2048
    return pl.pallas_call(
        body,
        grid=(TRAJ // blk,),
        in_specs=[
            pl.BlockSpec((blk, 2 * EMB), lambda i: (i, 0)),
            pl.BlockSpec((EMB, EMB), lambda i: (0, 0)),
            pl.BlockSpec((1, EMB), lambda i: (0, 0)),
        ],
        out_specs=pl.BlockSpec((1, EMB, blk), lambda i: (0, 0, i)),
        out_shape=jax.ShapeDtypeStruct((1, EMB, TRAJ), jnp.float32),
    )(g, W, b2d)


def kernel(acs, emb_table, W, b):
    tableT = jnp.transpose(emb_table)
    packed = _tc_pack(tableT)
    packed_flat = jnp.reshape(packed, (2 * PACKED_FLAT, HEMB))
    idx2d = jnp.reshape(acs.astype(jnp.int32), (TRAJ // CHUNK, CHUNK))
    g = _sc_gather(idx2d, packed_flat)
    out = _tc_linear(g, W, jnp.reshape(b, (1, EMB)))
    return jnp.transpose(out, (2, 0, 1))


# speedup vs baseline: 1.0046x; 1.0046x over previous
import functools

import jax
import jax.numpy as jnp
from jax import lax
from jax.experimental import pallas as pl
from jax.experimental.pallas import tpu as pltpu
from jax.experimental.pallas import tpu_sc as plsc

TRAJ = 16384
EMB = 64
HEMB = EMB // 2
VOCAB = 100000
NC = 2
NS = 16
NW = NC * NS
ROWS_PER_W = TRAJ // NW
CHUNK = 128
NCHUNK = ROWS_PER_W // CHUNK

SUB = 512
GROUP = 4 * SUB
TGRID = 49
PACKED_ROWS = TGRID * SUB
PACKED_FLAT = 4 * PACKED_ROWS


def _lrelu(y):
    return jnp.where(y >= 0, y, 0.01 * y)


def _rtne_bf16_bits(x):
    b = lax.bitcast_convert_type(x, jnp.uint32)
    return b + jnp.uint32(0x7FFF) + ((b >> 16) & jnp.uint32(1))


def _pack_bf16(ylo, yhi):
    lo = _rtne_bf16_bits(ylo) >> 16
    hi = _rtne_bf16_bits(yhi) & jnp.uint32(0xFFFF0000)
    return lax.bitcast_convert_type(lo | hi, jnp.float32)


def _tc_pack(tableT):
    def body(t0, t1, t2, t3, o_ref):
        row = lax.broadcasted_iota(jnp.int32, (4 * EMB, 2 * EMB), 0)
        col = lax.broadcasted_iota(jnp.int32, (4 * EMB, 2 * EMB), 1)
        same_sub = (row >> 6) == (col >> 5)
        d = row & 63
        j = col & 31
        sel_lo = (same_sub & (d == j)).astype(jnp.float32)
        sel_hi = (same_sub & (d == j + HEMB)).astype(jnp.float32)
        t = jnp.concatenate([t0[...], t1[...], t2[...], t3[...]], axis=0)
        ylo = lax.dot_general(t, sel_lo, (((0,), (0,)), ((), ())),
                              preferred_element_type=jnp.float32)
        yhi = lax.dot_general(t, sel_hi, (((0,), (0,)), ((), ())),
                              preferred_element_type=jnp.float32)
        o_ref[...] = _pack_bf16(ylo, yhi)

    return pl.pallas_call(
        body,
        grid=(TGRID,),
        in_specs=[
            pl.BlockSpec((EMB, SUB), lambda i, s=s: (0, 4 * i + s))
            for s in range(4)
        ],
        out_specs=pl.BlockSpec((SUB, 4 * HEMB), lambda i: (i, 0)),
        out_shape=jax.ShapeDtypeStruct((2 * PACKED_ROWS, 4 * HEMB), jnp.float32),
    )(tableT, tableT, tableT, tableT)


def _sc_gather(idx2d, packed_flat):
    mesh = plsc.VectorSubcoreMesh(core_axis_name="c", subcore_axis_name="s")

    @functools.partial(
        pl.kernel,
        mesh=mesh,
        out_type=jax.ShapeDtypeStruct((TRAJ, 2 * EMB), jnp.float32),
        scratch_types=[
            pltpu.VMEM((NCHUNK, CHUNK), jnp.int32),
            pltpu.VMEM((NCHUNK, CHUNK), jnp.int32),
            pltpu.VMEM((ROWS_PER_W, HEMB), jnp.float32),
            pltpu.SemaphoreType.DMA,
        ],
        compiler_params=pltpu.CompilerParams(use_tc_tiling_on_sc=False),
    )
    def k(idx_hbm, table_hbm, out_hbm, idx_raw, idx_t, rows_v, sem):
        wid = lax.axis_index("s") * NC + lax.axis_index("c")
        base = wid * ROWS_PER_W
        pltpu.sync_copy(idx_hbm.at[pl.ds(wid * NCHUNK, NCHUNK)], idx_raw)
        for c in range(NCHUNK):
            for kk in range(CHUNK // 16):
                v = idx_raw[c, pl.ds(kk * 16, 16)]
                grp = jnp.left_shift(jnp.right_shift(v, 11), 11)
                loc = jnp.left_shift(jnp.bitwise_and(v, jnp.int32(SUB - 1)), 2)
                sub = jnp.bitwise_and(jnp.right_shift(v, 9), jnp.int32(3))
                idx_t[c, pl.ds(kk * 16, 16)] = grp + loc + sub
        copies = [
            pltpu.make_async_copy(
                table_hbm.at[idx_t.at[c]],
                rows_v.at[pl.ds(c * CHUNK, CHUNK)],
                sem,
            )
            for c in range(NCHUNK)
        ]
        for cp in copies:
            cp.start()
        for cp in copies:
            cp.wait()
        pltpu.sync_copy(
            rows_v, out_hbm.at[pl.ds(base, ROWS_PER_W), pl.ds(0, HEMB)]
        )

    return k(idx2d, packed_flat)


def _tc_linear(g, W, b2d):
    def body(g_ref, w_ref, b_ref, o_ref):
        xi = lax.bitcast_convert_type(g_ref[:, 0:HEMB], jnp.uint32)
        xlo = lax.bitcast_convert_type(xi << 16, jnp.float32)
        xhi = lax.bitcast_convert_type(xi & jnp.uint32(0xFFFF0000), jnp.float32)
        ylo = lax.dot_general(w_ref[:, 0:HEMB], xlo, (((1,), (1,)), ((), ())),
                              preferred_element_type=jnp.float32)
        yhi = lax.dot_general(w_ref[:, HEMB:EMB], xhi, (((1,), (1,)), ((), ())),
                              preferred_element_type=jnp.float32)
        o_ref[0] = _lrelu(ylo + yhi + b_ref[...].T)

    blk = 2048
    return pl.pallas_call(
        body,
        grid=(TRAJ // blk,),
        in_specs=[
            pl.BlockSpec((blk, 2 * EMB), lambda i: (i, 0)),
            pl.BlockSpec((EMB, EMB), lambda i: (0, 0)),
            pl.BlockSpec((1, EMB), lambda i: (0, 0)),
        ],
        out_specs=pl.BlockSpec((1, EMB, blk), lambda i: (0, 0, i)),
        out_shape=jax.ShapeDtypeStruct((1, EMB, TRAJ), jnp.float32),
    )(g, W, b2d)


def kernel(acs, emb_table, W, b):
    tableT = jnp.transpose(emb_table)
    packed = _tc_pack(tableT)
    packed_flat = jnp.reshape(packed, (2 * PACKED_FLAT, HEMB))
    idx2d = jnp.reshape(acs.astype(jnp.int32), (TRAJ // CHUNK, CHUNK))
    g = _sc_gather(idx2d, packed_flat)
    out = _tc_linear(g, W, jnp.reshape(b, (1, EMB)))
    return jnp.transpose(out, (2, 0, 1))
